# Initial kernel scaffold; baseline (speedup 1.0000x reference)
import jax
import jax.numpy as jnp
from jax.experimental import pallas as pl


def kernel(x, flips, bp, cent):
    raise NotImplementedError("write your pallas kernel here")


# speedup vs baseline: 7.5352x; 7.5352x over previous
import functools

import jax
import jax.numpy as jnp
import numpy as np
from jax import lax
from jax.experimental import pallas as pl
from jax.experimental.pallas import tpu as pltpu

_D_IN = 768
_D = 1024
_L = 16
_CH = 256
_ROWS_PER_BLOCK = 512


def _hadamard_f32(n: int) -> np.ndarray:
    h = np.array([[1.0]], dtype=np.float32)
    while h.shape[0] < n:
        h = np.block([[h, h], [h, -h]])
    return h


def _body(x_ref, flips_ref, h_ref, bp_ref, cent_ref, o_ref):
    hs = h_ref[...]
    z = x_ref[...] * flips_ref[...]
    r = z.shape[0]
    z3 = z.reshape(r, 3, _CH)
    m = lax.dot_general(z3, hs, (((2,), (0,)), ((), ())),
                        precision=lax.Precision.HIGHEST,
                        preferred_element_type=jnp.float32)
    m0 = m[:, 0, :]
    m1 = m[:, 1, :]
    m2 = m[:, 2, :]
    b0 = m0 + m1
    b1 = m0 - m1
    y0 = b0 + m2
    y1 = b1 + m2
    y2 = b0 - m2
    y3 = b1 - m2

    def quant(y):
        q = jnp.full_like(y, cent_ref[0])
        for i in range(_L - 1):
            step = cent_ref[i + 1] - cent_ref[i]
            q = q + jnp.where(y > bp_ref[i], step, 0.0)
        return q

    q0 = quant(y0)
    q1 = quant(y1)
    q2 = quant(y2)
    q3 = quant(y3)

    u0 = q0 + q1
    u1 = q0 - q1
    u2 = q2 + q3
    u3 = q2 - q3
    p = jnp.concatenate(
        [(u0 + u2)[:, None, :], (u1 + u3)[:, None, :], (u0 - u2)[:, None, :]],
        axis=1)
    deq = lax.dot_general(p, hs, (((2,), (0,)), ((), ())),
                          preferred_element_type=jnp.float32)
    o_ref[...] = deq.reshape(r, _D_IN) * flips_ref[...]


@jax.jit
def kernel(x, flips, bp, cent):
    orig_dtype = x.dtype
    n = x.shape[0] * x.shape[1]
    xf = x.reshape(n, _D_IN).astype(jnp.float32)
    hs = jnp.asarray(_hadamard_f32(_CH) * (1.0 / 32.0))
    flips768 = flips[:_D_IN].reshape(1, _D_IN)

    r = _ROWS_PER_BLOCK
    grid = (n // r,)
    out = pl.pallas_call(
        _body,
        grid=grid,
        in_specs=[
            pl.BlockSpec((r, _D_IN), lambda i: (i, 0)),
            pl.BlockSpec((1, _D_IN), lambda i: (0, 0)),
            pl.BlockSpec((_CH, _CH), lambda i: (0, 0)),
            pl.BlockSpec(memory_space=pltpu.SMEM),
            pl.BlockSpec(memory_space=pltpu.SMEM),
        ],
        out_specs=pl.BlockSpec((r, _D_IN), lambda i: (i, 0)),
        out_shape=jax.ShapeDtypeStruct((n, _D_IN), jnp.float32),
        compiler_params=pltpu.CompilerParams(
            dimension_semantics=("arbitrary",)),
    )(xf, flips768, hs, bp, cent)
    return out.reshape(x.shape).astype(orig_dtype)
